# Initial kernel scaffold; baseline (speedup 1.0000x reference)
import functools

import jax
import jax.numpy as jnp
from jax.experimental import pallas as pl
from jax.experimental.pallas import tpu as pltpu

N_NODES = 5000
D = 128
N_LAYERS_ = 3
R = 250
NBLK = N_NODES // R


def _sweep_kernel(a_ref, x0_ref, y0_ref, users_ref, items_ref,
                  x_cur, s1, s2, acc, invu, vdeg, invv):
    l = pl.program_id(0)
    i = pl.program_id(1)

    @pl.when((l == 0) & (i == 0))
    def _init():
        x_cur[...] = x0_ref[...]
        acc[...] = x0_ref[...]

    @pl.when(i == 0)
    def _zero_s2():
        s2[...] = jnp.zeros_like(s2)

    a = a_ref[...]
    x = x_cur[...]
    s1[pl.ds(i * R, R), :] = jax.lax.dot_general(
        a, x, (((1,), (0,)), ((), ())), preferred_element_type=jnp.float32)
    xi = x_cur[pl.ds(i * R, R), :]
    s2[...] += jax.lax.dot_general(
        a, xi, (((0,), (0,)), ((), ())), preferred_element_type=jnp.float32)

    @pl.when(l == 0)
    def _degrees():
        ones_n = jnp.ones((N_NODES, 1), jnp.float32)
        udeg_blk = jax.lax.dot_general(
            a, ones_n, (((1,), (0,)), ((), ())),
            preferred_element_type=jnp.float32)
        invu[pl.ds(i * R, R), :] = udeg_blk
        ones_r = jnp.ones((R, 1), jnp.float32)

        @pl.when(i == 0)
        def _zv():
            vdeg[...] = jnp.zeros_like(vdeg)

        vdeg[...] += jax.lax.dot_general(
            a, ones_r, (((0,), (0,)), ((), ())),
            preferred_element_type=jnp.float32)

    @pl.when((l == 0) & (i == NBLK - 1))
    def _inv():
        invu[...] = 1.0 / (invu[...] + 1.0)
        invv[...] = 1.0 / (vdeg[...] + 1.0)

    @pl.when(i == NBLK - 1)
    def _finalize_layer():
        x_new = (x_cur[...] + s1[...]) * invu[...] + s2[...] * invv[...]
        x_cur[...] = x_new
        acc[...] += x_new

    @pl.when((l == N_LAYERS_ - 1) & (i == NBLK - 1))
    def _emit():
        users_ref[...] = acc[...] * 0.25
        iv = invv[...]
        scale = (1.0 + iv + iv * iv + iv * iv * iv) * 0.25
        items_ref[...] = y0_ref[...] * scale


@functools.partial(jax.jit, static_argnames=("interpret",))
def _run(a, x0, y0, interpret=False):
    return pl.pallas_call(
        _sweep_kernel,
        grid=(N_LAYERS_, NBLK),
        in_specs=[
            pl.BlockSpec((R, N_NODES), lambda l, i: (i, 0)),
            pl.BlockSpec((N_NODES, D), lambda l, i: (0, 0)),
            pl.BlockSpec((N_NODES, D), lambda l, i: (0, 0)),
        ],
        out_specs=[
            pl.BlockSpec((N_NODES, D), lambda l, i: (0, 0)),
            pl.BlockSpec((N_NODES, D), lambda l, i: (0, 0)),
        ],
        out_shape=[
            jax.ShapeDtypeStruct((N_NODES, D), jnp.float32),
            jax.ShapeDtypeStruct((N_NODES, D), jnp.float32),
        ],
        scratch_shapes=[
            pltpu.VMEM((N_NODES, D), jnp.float32),
            pltpu.VMEM((N_NODES, D), jnp.float32),
            pltpu.VMEM((N_NODES, D), jnp.float32),
            pltpu.VMEM((N_NODES, D), jnp.float32),
            pltpu.VMEM((N_NODES, 1), jnp.float32),
            pltpu.VMEM((N_NODES, 1), jnp.float32),
            pltpu.VMEM((N_NODES, 1), jnp.float32),
        ],
        interpret=interpret,
    )(a, x0, y0)


def kernel(edge_index, user_embedding, item_embedding):
    u = edge_index[0].astype(jnp.int32)
    v = edge_index[1].astype(jnp.int32)
    a = jnp.zeros((N_NODES, N_NODES), jnp.float32).at[u, v].add(1.0)
    users, items = _run(a, user_embedding, item_embedding)
    return (users, items)


# speedup vs baseline: 16.7556x; 16.7556x over previous
import functools

import jax
import jax.numpy as jnp
from jax.experimental import pallas as pl
from jax.experimental.pallas import tpu as pltpu

N_NODES = 5000
D = 128
N_LAYERS_ = 3
R = 200
NBLK = N_NODES // R


def _sweep_kernel(a_ref, x0_ref, y0_ref, users_ref, items_ref,
                  x_cur, s1, s2, acc, invu, vdeg, invv):
    l = pl.program_id(0)
    i = pl.program_id(1)

    @pl.when((l == 0) & (i == 0))
    def _init():
        x_cur[...] = x0_ref[...]
        acc[...] = x0_ref[...]

    @pl.when(i == 0)
    def _zero_s2():
        s2[...] = jnp.zeros_like(s2)

    a = a_ref[...]
    x = x_cur[...]
    s1[pl.ds(i * R, R), :] = jax.lax.dot_general(
        a, x, (((1,), (0,)), ((), ())), preferred_element_type=jnp.float32)
    xi = x_cur[pl.ds(i * R, R), :]
    s2[...] += jax.lax.dot_general(
        a, xi, (((0,), (0,)), ((), ())), preferred_element_type=jnp.float32)

    @pl.when(l == 0)
    def _degrees():
        ones_n = jnp.ones((N_NODES, 1), jnp.float32)
        udeg_blk = jax.lax.dot_general(
            a, ones_n, (((1,), (0,)), ((), ())),
            preferred_element_type=jnp.float32)
        invu[pl.ds(i * R, R), :] = udeg_blk
        ones_r = jnp.ones((R, 1), jnp.float32)

        @pl.when(i == 0)
        def _zv():
            vdeg[...] = jnp.zeros_like(vdeg)

        vdeg[...] += jax.lax.dot_general(
            a, ones_r, (((0,), (0,)), ((), ())),
            preferred_element_type=jnp.float32)

    @pl.when((l == 0) & (i == NBLK - 1))
    def _inv():
        invu[...] = 1.0 / (invu[...] + 1.0)
        invv[...] = 1.0 / (vdeg[...] + 1.0)

    @pl.when(i == NBLK - 1)
    def _finalize_layer():
        x_new = (x_cur[...] + s1[...]) * invu[...] + s2[...] * invv[...]
        x_cur[...] = x_new
        acc[...] += x_new

    @pl.when((l == N_LAYERS_ - 1) & (i == NBLK - 1))
    def _emit():
        users_ref[...] = acc[...] * 0.25
        iv = invv[...]
        scale = (1.0 + iv + iv * iv + iv * iv * iv) * 0.25
        items_ref[...] = y0_ref[...] * scale


@functools.partial(jax.jit, static_argnames=("interpret",))
def _run(a, x0, y0, interpret=False):
    return pl.pallas_call(
        _sweep_kernel,
        grid=(N_LAYERS_, NBLK),
        in_specs=[
            pl.BlockSpec((R, N_NODES), lambda l, i: (i, 0)),
            pl.BlockSpec((N_NODES, D), lambda l, i: (0, 0)),
            pl.BlockSpec((N_NODES, D), lambda l, i: (0, 0)),
        ],
        out_specs=[
            pl.BlockSpec((N_NODES, D), lambda l, i: (0, 0)),
            pl.BlockSpec((N_NODES, D), lambda l, i: (0, 0)),
        ],
        out_shape=[
            jax.ShapeDtypeStruct((N_NODES, D), jnp.float32),
            jax.ShapeDtypeStruct((N_NODES, D), jnp.float32),
        ],
        scratch_shapes=[
            pltpu.VMEM((N_NODES, D), jnp.float32),
            pltpu.VMEM((N_NODES, D), jnp.float32),
            pltpu.VMEM((N_NODES, D), jnp.float32),
            pltpu.VMEM((N_NODES, D), jnp.float32),
            pltpu.VMEM((N_NODES, 1), jnp.float32),
            pltpu.VMEM((N_NODES, 1), jnp.float32),
            pltpu.VMEM((N_NODES, 1), jnp.float32),
        ],
        interpret=interpret,
    )(a, x0, y0)


def kernel(edge_index, user_embedding, item_embedding):
    u = edge_index[0].astype(jnp.int32)
    v = edge_index[1].astype(jnp.int32)
    a = jnp.zeros((N_NODES, N_NODES), jnp.float32).at[u, v].add(1.0)
    users, items = _run(a, user_embedding, item_embedding)
    return (users, items)
